# baseline (speedup 1.0000x reference)
import jax
import jax.numpy as jnp
from jax import lax
from jax.experimental import pallas as pl
from jax.experimental.pallas import tpu as pltpu
from jax.experimental.pallas import tpu_sc as plsc

B = 16384
R = 32
L = 16
NC = 2
NS = 16
NW = NC * NS
BPW = B // NW
GCHUNK = 128
UNROLL = 8


def _cpd_body(idx0_hbm, idx1_hbm, idx2_hbm, e0_hbm, e1_hbm, e2_hbm, out_hbm,
              idx0_v, idx1_v, idx2_v, r0_v, r1_v, r2_v, sums_v, sem):
    wid = lax.axis_index("s") * NC + lax.axis_index("c")
    base = wid * BPW

    pltpu.sync_copy(idx0_hbm.at[pl.ds(base, BPW)], idx0_v)
    pltpu.sync_copy(idx1_hbm.at[pl.ds(base, BPW)], idx1_v)
    pltpu.sync_copy(idx2_hbm.at[pl.ds(base, BPW)], idx2_v)

    copies = []
    for e_hbm, idx_v, r_v in ((e0_hbm, idx0_v, r0_v),
                              (e1_hbm, idx1_v, r1_v),
                              (e2_hbm, idx2_v, r2_v)):
        for k in range(BPW // GCHUNK):
            copies.append(pltpu.async_copy(
                e_hbm.at[idx_v.at[pl.ds(k * GCHUNK, GCHUNK)]],
                r_v.at[pl.ds(k * GCHUNK, GCHUNK), :],
                sem))
    for c in copies:
        c.wait()

    def row_body(i0, carry):
        for j in range(UNROLL):
            i = i0 * UNROLL + j
            a = (r0_v[i, pl.ds(0, L)] * r1_v[i, pl.ds(0, L)]
                 * r2_v[i, pl.ds(0, L)])
            b = (r0_v[i, pl.ds(L, L)] * r1_v[i, pl.ds(L, L)]
                 * r2_v[i, pl.ds(L, L)])
            sums_v[i, :] = a + b
        return carry
    lax.fori_loop(0, BPW // UNROLL, row_body, 0)

    pltpu.sync_copy(sums_v, out_hbm.at[pl.ds(base, BPW), :])


def _rank_fold_sc(idx0, idx1, idx2, E0, E1, E2):
    run = pl.kernel(
        _cpd_body,
        out_type=jax.ShapeDtypeStruct((B, L), jnp.float32),
        mesh=plsc.VectorSubcoreMesh(core_axis_name="c", subcore_axis_name="s"),
        compiler_params=pltpu.CompilerParams(use_tc_tiling_on_sc=False),
        scratch_types=[
            pltpu.VMEM((BPW,), jnp.int32),
            pltpu.VMEM((BPW,), jnp.int32),
            pltpu.VMEM((BPW,), jnp.int32),
            pltpu.VMEM((BPW, R), jnp.float32),
            pltpu.VMEM((BPW, R), jnp.float32),
            pltpu.VMEM((BPW, R), jnp.float32),
            pltpu.VMEM((BPW, L), jnp.float32),
            pltpu.SemaphoreType.DMA,
        ],
    )
    return run(idx0, idx1, idx2, E0, E1, E2)


def _lane_sum_body(p_ref, o_ref):
    c = lax.broadcasted_iota(jnp.int32, (128, 8), 0)
    k = lax.broadcasted_iota(jnp.int32, (128, 8), 1)
    m = jnp.where(c // L == k, 1.0, 0.0).astype(jnp.float32)
    o_ref[:] = jnp.dot(p_ref[:], m, preferred_element_type=jnp.float32)


def _lane_sum_tc(partials):
    folded = pl.pallas_call(
        _lane_sum_body,
        out_shape=jax.ShapeDtypeStruct((B // 8, 8), jnp.float32),
    )(partials.reshape(B // 8, 128))
    return folded.reshape(B)


@jax.jit
def kernel(idxs, E0, E1, E2):
    idx0 = idxs[:, 0].astype(jnp.int32)
    idx1 = idxs[:, 1].astype(jnp.int32)
    idx2 = idxs[:, 2].astype(jnp.int32)
    partials = _rank_fold_sc(idx0, idx1, idx2, E0, E1, E2)
    return _lane_sum_tc(partials)
